# baseline (speedup 1.0000x reference)
import functools

import jax
import jax.numpy as jnp
from jax import lax
from jax.experimental import pallas as pl
from jax.experimental.pallas import tpu as pltpu
from jax.experimental.pallas import tpu_sc as plsc

C = 128
G = 2
NUM_CORES = 2
NUM_SUBCORES = 16
NW = NUM_CORES * NUM_SUBCORES


def _transform_kernel(x_ref, w_ref, o_ref):
    o_ref[...] = lax.dot_general(
        x_ref[...], w_ref[0],
        dimension_numbers=(((1,), (1,)), ((), ())),
        preferred_element_type=jnp.float32,
    )


def _combine_kernel(p0_ref, p1_ref, b_ref, o_ref):
    o_ref[...] = p0_ref[0] + p1_ref[0] + b_ref[...]


def _sc_body(n_pad, n_nodes, per_core_groups, per_sub_groups, rows_per_sub,
             h_ref, src_ref, et_ref, dst_ref, z_ref, out_ref,
             srcv, etv, gv, dstv, rows, acc, isem, gsem):
    c = lax.axis_index("c")
    s = lax.axis_index("s")
    row0 = s * rows_per_sub
    pltpu.sync_copy(z_ref.at[pl.ds(row0, rows_per_sub)],
                    acc.at[pl.ds(row0, rows_per_sub)])
    plsc.subcore_barrier()

    base_group = c * per_core_groups + s * per_sub_groups

    @pl.loop(0, per_sub_groups)
    def _(j):
        gi = base_group + j
        cp_s = pltpu.async_copy(src_ref.at[gi], srcv, isem)
        cp_e = pltpu.async_copy(et_ref.at[gi], etv, isem)
        cp_d = pltpu.async_copy(dst_ref.at[gi], dstv, isem)
        cp_s.wait()
        cp_e.wait()
        cp_d.wait()
        for k in range(G):
            for m in range(C // 16):
                sl = pl.ds(m * 16, 16)
                gv[k, sl] = etv[k, sl] * n_nodes + srcv[k, sl]
        cps = [pltpu.async_copy(h_ref.at[gv.at[k]], rows.at[k], gsem)
               for k in range(G)]
        for cp in cps:
            cp.wait()
        cps = [pltpu.async_copy(rows.at[k], acc.at[dstv.at[k]], isem, add=True)
               for k in range(G)]
        for cp in cps:
            cp.wait()

    plsc.subcore_barrier()
    pltpu.sync_copy(acc.at[pl.ds(row0, rows_per_sub)],
                    out_ref.at[c, pl.ds(row0, rows_per_sub)])


def kernel(node_features, node_repr, edge_index, edge_types, num_relations,
           weight, bias):
    del node_features, num_relations
    n = node_repr.shape[0]
    d_in = node_repr.shape[1]
    r = weight.shape[0]
    d_out = weight.shape[1]
    e = edge_types.shape[0]

    bn = 1000
    assert n % bn == 0
    h_all = pl.pallas_call(
        _transform_kernel,
        grid=(r, n // bn),
        in_specs=[
            pl.BlockSpec((bn, d_in), lambda ri, ni: (ni, 0)),
            pl.BlockSpec((1, d_out, d_in), lambda ri, ni: (ri, 0, 0)),
        ],
        out_specs=pl.BlockSpec((bn, d_out), lambda ri, ni: (ri * (n // bn) + ni, 0)),
        out_shape=jax.ShapeDtypeStruct((r * n, d_out), jnp.float32),
    )(node_repr, weight)

    groups_total = -(-e // (G * C * NW)) * NW
    e_pad = groups_total * G * C
    pad = e_pad - e
    src_p = jnp.concatenate(
        [edge_index[0], jnp.zeros((pad,), jnp.int32)]).reshape(groups_total, G, C)
    et_p = jnp.concatenate(
        [edge_types, jnp.zeros((pad,), jnp.int32)]).reshape(groups_total, G, C)
    dst_p = jnp.concatenate(
        [edge_index[1], jnp.full((pad,), n, jnp.int32)]).reshape(groups_total, G, C)

    per_core_groups = groups_total // NUM_CORES
    per_sub_groups = per_core_groups // NUM_SUBCORES
    rows_per_sub = -(-(n + 1) // (NUM_SUBCORES * 8)) * 8
    n_pad = rows_per_sub * NUM_SUBCORES
    zeros_init = jnp.zeros((n_pad, d_out), jnp.float32)

    mesh = plsc.VectorSubcoreMesh(core_axis_name="c", subcore_axis_name="s")
    sc_kernel = pl.kernel(
        functools.partial(_sc_body, n_pad, n, per_core_groups, per_sub_groups,
                          rows_per_sub),
        out_type=jax.ShapeDtypeStruct((NUM_CORES, n_pad, d_out), jnp.float32),
        mesh=mesh,
        scratch_types=[
            pltpu.VMEM((G, C), jnp.int32),
            pltpu.VMEM((G, C), jnp.int32),
            pltpu.VMEM((G, C), jnp.int32),
            pltpu.VMEM((G, C), jnp.int32),
            pltpu.VMEM((G, C, 128), jnp.float32),
            pltpu.VMEM_SHARED((n_pad, 128), jnp.float32),
            pltpu.SemaphoreType.DMA,
            pltpu.SemaphoreType.DMA,
        ],
    )
    partials = sc_kernel(h_all, src_p, et_p, dst_p, zeros_init)

    out = pl.pallas_call(
        _combine_kernel,
        grid=(n // bn,),
        in_specs=[
            pl.BlockSpec((1, bn, d_out), lambda i: (0, i, 0)),
            pl.BlockSpec((1, bn, d_out), lambda i: (1, i, 0)),
            pl.BlockSpec((1, d_out), lambda i: (0, 0)),
        ],
        out_specs=pl.BlockSpec((bn, d_out), lambda i: (i, 0)),
        out_shape=jax.ShapeDtypeStruct((n, d_out), jnp.float32),
    )(partials, partials, bias.reshape(1, d_out))

    return out
